# baseline (speedup 1.0000x reference)
import functools

import jax
import jax.numpy as jnp
from jax import lax
from jax.experimental import pallas as pl
from jax.experimental.pallas import tpu as pltpu
from jax.experimental.pallas import tpu_sc as plsc

N = 10000
E = 320000
D_IN = 128
D_H = 128
D_OUT = 64

NC = 2
NS = 16
NW = NC * NS
EPW = E // NW
K = 80
NCH = EPW // K
KE = 96
EPWP = 10176
NCHE = EPWP // KE
NPAD = 10240
RPT = NPAD // NS

@functools.lru_cache(maxsize=None)
def _sc_kernels():
    mesh = plsc.VectorSubcoreMesh(core_axis_name="c", subcore_axis_name="s")

    @functools.partial(
        pl.kernel,
        mesh=mesh,
        out_type=jax.ShapeDtypeStruct((NC, N), jnp.float32),
        scratch_types=[
            pltpu.VMEM((NCH, K), jnp.int32),
            pltpu.VMEM((K,), jnp.float32),
            pltpu.VMEM_SHARED((N,), jnp.float32),
        ],
    )
    def deg_kernel(row_hbm, zeros_hbm, out_hbm, idx_v, ones_v, acc_sh):
        c = lax.axis_index("c")
        s = lax.axis_index("s")
        wid = c * NS + s
        pltpu.sync_copy(row_hbm.at[wid], idx_v)
        for i in range(K // 16):
            ones_v[pl.ds(i * 16, 16)] = jnp.ones((16,), jnp.float32)

        @pl.when(s == 0)
        def _zero():
            pltpu.sync_copy(zeros_hbm, acc_sh)

        plsc.subcore_barrier()

        def body(j, carry):
            pltpu.sync_copy(ones_v, acc_sh.at[idx_v.at[j]], add=True)
            return carry

        lax.fori_loop(0, NCH, body, 0)
        plsc.subcore_barrier()

        @pl.when(s == 0)
        def _writeback():
            pltpu.sync_copy(acc_sh, out_hbm.at[c])

    @functools.partial(
        pl.kernel,
        mesh=mesh,
        out_type=jax.ShapeDtypeStruct((NC, NPAD, D_H), jnp.float32),
        scratch_types=[
            pltpu.VMEM((EPWP,), jnp.int32),
            pltpu.VMEM((NCHE, KE), jnp.int32),
            pltpu.VMEM((KE, D_H), jnp.float32),
            pltpu.VMEM((KE, D_H), jnp.float32),
            pltpu.VMEM_SHARED((NPAD, D_H), jnp.float32),
            pltpu.SemaphoreType.DMA,
            pltpu.SemaphoreType.DMA,
        ],
    )
    def edge_kernel(g_hbm, col_hbm, row_hbm, zeros_hbm, out_hbm,
                    colv, rowv, buf0, buf1, acc_sh, sem0, sem1):
        c = lax.axis_index("c")
        s = lax.axis_index("s")
        wid = c * NS + s
        pltpu.sync_copy(col_hbm.at[wid], colv)
        pltpu.sync_copy(row_hbm.at[wid], rowv)
        pltpu.sync_copy(zeros_hbm, acc_sh.at[pl.ds(s * RPT, RPT)])
        plsc.subcore_barrier()

        def body(i, carry):
            g = 2 * i
            h0 = pltpu.async_copy(
                g_hbm.at[colv.at[pl.ds(g * KE, KE)]], buf0, sem0)
            h1 = pltpu.async_copy(
                g_hbm.at[colv.at[pl.ds((g + 1) * KE, KE)]], buf1, sem1)
            h0.wait()
            pltpu.sync_copy(buf0, acc_sh.at[rowv.at[g]], add=True)
            h1.wait()
            pltpu.sync_copy(buf1, acc_sh.at[rowv.at[g + 1]], add=True)
            return carry

        lax.fori_loop(0, NCHE // 2, body, 0)
        plsc.subcore_barrier()
        pltpu.sync_copy(acc_sh.at[pl.ds(s * RPT, RPT)],
                        out_hbm.at[c, pl.ds(s * RPT, RPT)])

    return deg_kernel, edge_kernel


BN = 2000


def _tc1_body(x_ref, w_ref, b_ref, d0_ref, d1_ref, g_ref, dinv_ref):
    d = d0_ref[...] + d1_ref[...] + 1.0
    dinv = lax.rsqrt(d)
    dinv_ref[...] = dinv
    h = jnp.dot(x_ref[...], w_ref[...], preferred_element_type=jnp.float32)
    g_ref[...] = (h + b_ref[...]) * dinv


def _tc2_body(p0_ref, p1_ref, g_ref, dinv_ref, w_ref, b_ref, out_ref):
    dinv = dinv_ref[...]
    h = jax.nn.relu((p0_ref[...] + p1_ref[...] + g_ref[...]) * dinv)
    h = jnp.dot(h, w_ref[...], preferred_element_type=jnp.float32)
    out_ref[...] = (h + b_ref[...]) * dinv


def _tc3_body(p0_ref, p1_ref, g_ref, dinv_ref, w1_ref, b1_ref,
              w2_ref, b2_ref, out_ref):
    dinv = dinv_ref[...]
    h = jax.nn.relu((p0_ref[...] + p1_ref[...] + g_ref[...]) * dinv)
    nrm = jnp.maximum(jnp.sqrt(jnp.sum(h * h, axis=1, keepdims=True)), 1e-12)
    h = h / nrm
    h = jnp.dot(h, w1_ref[...], preferred_element_type=jnp.float32) + b1_ref[...]
    z = jnp.dot(h, w2_ref[...], preferred_element_type=jnp.float32) + b2_ref[...]
    m = jnp.max(z, axis=1, keepdims=True)
    zz = z - m
    out_ref[...] = zz - jnp.log(jnp.sum(jnp.exp(zz), axis=1, keepdims=True))


def _row_blk(d):
    return pl.BlockSpec((BN, d), lambda i: (i, 0))


def _full_blk(r, c):
    return pl.BlockSpec((r, c), lambda i: (0, 0))


def kernel(x, edge_index, W1, b1, W2, b2, Wp1, bp1, Wp2, bp2):
    row_deg = edge_index[0].reshape(NW, NCH, K)
    npad_e = EPWP - EPW
    sink = N + ((jnp.arange(npad_e, dtype=jnp.int32)[None, :]
                 + 8 * jnp.arange(NW, dtype=jnp.int32)[:, None])
                % max(npad_e, 1))
    rowp = jnp.concatenate(
        [edge_index[0].reshape(NW, EPW), sink], axis=1)
    colp = jnp.concatenate(
        [edge_index[1].reshape(NW, EPW),
         jnp.zeros((NW, npad_e), jnp.int32)], axis=1)
    rowp = rowp.reshape(NW, NCHE, KE)
    zeros_n = jnp.zeros((N,), jnp.float32)
    zeros_nd = jnp.zeros((RPT, D_H), jnp.float32)
    _deg_kernel, _edge_kernel = _sc_kernels()

    deg_p = _deg_kernel(row_deg, zeros_n)
    d0 = deg_p[0].reshape(N, 1)
    d1 = deg_p[1].reshape(N, 1)

    g1, dinv = pl.pallas_call(
        _tc1_body,
        grid=(N // BN,),
        in_specs=[_row_blk(D_IN), _full_blk(D_IN, D_H), _full_blk(1, D_H),
                  _row_blk(1), _row_blk(1)],
        out_specs=[_row_blk(D_H), _row_blk(1)],
        out_shape=[jax.ShapeDtypeStruct((N, D_H), jnp.float32),
                   jax.ShapeDtypeStruct((N, 1), jnp.float32)],
    )(x, W1, b1.reshape(1, D_H), d0, d1)

    acc1 = _edge_kernel(g1, colp, rowp, zeros_nd)[:, :N, :]

    g2 = pl.pallas_call(
        _tc2_body,
        grid=(N // BN,),
        in_specs=[_row_blk(D_H), _row_blk(D_H), _row_blk(D_H), _row_blk(1),
                  _full_blk(D_H, D_H), _full_blk(1, D_H)],
        out_specs=_row_blk(D_H),
        out_shape=jax.ShapeDtypeStruct((N, D_H), jnp.float32),
    )(acc1[0], acc1[1], g1, dinv, W2, b2.reshape(1, D_H))

    acc2 = _edge_kernel(g2, colp, rowp, zeros_nd)[:, :N, :]

    out = pl.pallas_call(
        _tc3_body,
        grid=(N // BN,),
        in_specs=[_row_blk(D_H), _row_blk(D_H), _row_blk(D_H), _row_blk(1),
                  _full_blk(D_H, D_H), _full_blk(1, D_H),
                  _full_blk(D_H, D_OUT), _full_blk(1, D_OUT)],
        out_specs=_row_blk(D_OUT),
        out_shape=jax.ShapeDtypeStruct((N, D_OUT), jnp.float32),
    )(acc2[0], acc2[1], g2, dinv, Wp1, bp1.reshape(1, D_H),
      Wp2, bp2.reshape(1, D_OUT))

    return out


# speedup vs baseline: 1.9683x; 1.9683x over previous
import functools

import jax
import jax.numpy as jnp
from jax import lax
from jax.experimental import pallas as pl
from jax.experimental.pallas import tpu as pltpu
from jax.experimental.pallas import tpu_sc as plsc

N = 10000
E = 320000
D_IN = 128
D_H = 128
D_OUT = 64

NC = 2
NS = 16
NW = NC * NS
EPW = E // NW
K = 80
NCH = EPW // K
KE = 96
EPWP = 10176
NCHE = EPWP // KE
NPAD = 10240
RPT = NPAD // NS

@functools.lru_cache(maxsize=None)
def _sc_kernels():
    mesh = plsc.VectorSubcoreMesh(core_axis_name="c", subcore_axis_name="s")

    @functools.partial(
        pl.kernel,
        mesh=mesh,
        out_type=jax.ShapeDtypeStruct((NC, N), jnp.float32),
        scratch_types=[
            pltpu.VMEM((NCH, K), jnp.int32),
            pltpu.VMEM((K,), jnp.float32),
            pltpu.VMEM_SHARED((N,), jnp.float32),
        ],
    )
    def deg_kernel(row_hbm, zeros_hbm, out_hbm, idx_v, ones_v, acc_sh):
        c = lax.axis_index("c")
        s = lax.axis_index("s")
        wid = c * NS + s
        pltpu.sync_copy(row_hbm.at[wid], idx_v)
        for i in range(K // 16):
            ones_v[pl.ds(i * 16, 16)] = jnp.ones((16,), jnp.float32)

        @pl.when(s == 0)
        def _zero():
            pltpu.sync_copy(zeros_hbm, acc_sh)

        plsc.subcore_barrier()

        def body(j, carry):
            pltpu.sync_copy(ones_v, acc_sh.at[idx_v.at[j]], add=True)
            return carry

        lax.fori_loop(0, NCH, body, 0)
        plsc.subcore_barrier()

        @pl.when(s == 0)
        def _writeback():
            pltpu.sync_copy(acc_sh, out_hbm.at[c])

    @functools.partial(
        pl.kernel,
        mesh=mesh,
        out_type=jax.ShapeDtypeStruct((NC, NPAD, D_H), jnp.float32),
        scratch_types=[
            pltpu.VMEM((EPWP,), jnp.int32),
            pltpu.VMEM((NCHE, KE), jnp.int32),
            pltpu.VMEM((KE, D_H), jnp.float32),
            pltpu.VMEM((KE, D_H), jnp.float32),
            pltpu.VMEM_SHARED((NPAD, D_H), jnp.float32),
            pltpu.SemaphoreType.DMA,
            pltpu.SemaphoreType.DMA,
        ],
    )
    def edge_kernel(g_hbm, col_hbm, row_hbm, zeros_hbm, out_hbm,
                    colv, rowv, buf0, buf1, acc_sh, sem0, sem1):
        c = lax.axis_index("c")
        s = lax.axis_index("s")
        wid = c * NS + s
        pltpu.sync_copy(col_hbm.at[wid], colv)
        pltpu.sync_copy(row_hbm.at[wid], rowv)
        pltpu.sync_copy(zeros_hbm, acc_sh.at[pl.ds(s * RPT, RPT)])
        plsc.subcore_barrier()

        def body(i, carry):
            g = 2 * i
            h0 = pltpu.async_copy(
                g_hbm.at[colv.at[pl.ds(g * KE, KE)]], buf0, sem0)
            h1 = pltpu.async_copy(
                g_hbm.at[colv.at[pl.ds((g + 1) * KE, KE)]], buf1, sem1)
            h0.wait()
            pltpu.sync_copy(buf0, acc_sh.at[rowv.at[g]], add=True)
            h1.wait()
            pltpu.sync_copy(buf1, acc_sh.at[rowv.at[g + 1]], add=True)
            return carry

        lax.fori_loop(0, NCHE // 2, body, 0)
        plsc.subcore_barrier()
        pltpu.sync_copy(acc_sh.at[pl.ds(s * RPT, RPT)],
                        out_hbm.at[c, pl.ds(s * RPT, RPT)])

    return deg_kernel, edge_kernel


BN = 2000


def _tc1_body(x_ref, w_ref, b_ref, d0_ref, d1_ref, g_ref, dinv_ref):
    d = d0_ref[...] + d1_ref[...] + 1.0
    dinv = lax.rsqrt(d)
    dinv_ref[...] = dinv
    h = jnp.dot(x_ref[...], w_ref[...], preferred_element_type=jnp.float32)
    g_ref[...] = (h + b_ref[...]) * dinv


def _tc2_body(p0_ref, p1_ref, g_ref, dinv_ref, w_ref, b_ref, out_ref):
    dinv = dinv_ref[...]
    h = jax.nn.relu((p0_ref[...] + p1_ref[...] + g_ref[...]) * dinv)
    h = jnp.dot(h, w_ref[...], preferred_element_type=jnp.float32)
    out_ref[...] = (h + b_ref[...]) * dinv


def _tc3_body(p0_ref, p1_ref, g_ref, dinv_ref, w1_ref, b1_ref,
              w2_ref, b2_ref, out_ref):
    dinv = dinv_ref[...]
    h = jax.nn.relu((p0_ref[...] + p1_ref[...] + g_ref[...]) * dinv)
    nrm = jnp.maximum(jnp.sqrt(jnp.sum(h * h, axis=1, keepdims=True)), 1e-12)
    h = h / nrm
    h = jnp.dot(h, w1_ref[...], preferred_element_type=jnp.float32) + b1_ref[...]
    z = jnp.dot(h, w2_ref[...], preferred_element_type=jnp.float32) + b2_ref[...]
    m = jnp.max(z, axis=1, keepdims=True)
    zz = z - m
    out_ref[...] = zz - jnp.log(jnp.sum(jnp.exp(zz), axis=1, keepdims=True))


def _row_blk(d):
    return pl.BlockSpec((BN, d), lambda i: (i, 0))


def _full_blk(r, c):
    return pl.BlockSpec((r, c), lambda i: (0, 0))


def kernel(x, edge_index, W1, b1, W2, b2, Wp1, bp1, Wp2, bp2):
    row_deg = edge_index[0].reshape(NW, NCH, K)
    npad_e = EPWP - EPW
    sink = N + ((jnp.arange(npad_e, dtype=jnp.int32)[None, :]
                 + 8 * jnp.arange(NW, dtype=jnp.int32)[:, None])
                % max(npad_e, 1))
    rowp = jnp.concatenate(
        [edge_index[0].reshape(NW, EPW), sink], axis=1)
    dcol = ((jnp.arange(npad_e, dtype=jnp.int32)[None, :] * 97
             + 311 * jnp.arange(NW, dtype=jnp.int32)[:, None]) % N)
    colp = jnp.concatenate(
        [edge_index[1].reshape(NW, EPW), dcol], axis=1)
    rowp = rowp.reshape(NW, NCHE, KE)
    zeros_n = jnp.zeros((N,), jnp.float32)
    zeros_nd = jnp.zeros((RPT, D_H), jnp.float32)
    _deg_kernel, _edge_kernel = _sc_kernels()

    deg_p = _deg_kernel(row_deg, zeros_n)
    d0 = deg_p[0].reshape(N, 1)
    d1 = deg_p[1].reshape(N, 1)

    g1, dinv = pl.pallas_call(
        _tc1_body,
        grid=(N // BN,),
        in_specs=[_row_blk(D_IN), _full_blk(D_IN, D_H), _full_blk(1, D_H),
                  _row_blk(1), _row_blk(1)],
        out_specs=[_row_blk(D_H), _row_blk(1)],
        out_shape=[jax.ShapeDtypeStruct((N, D_H), jnp.float32),
                   jax.ShapeDtypeStruct((N, 1), jnp.float32)],
    )(x, W1, b1.reshape(1, D_H), d0, d1)

    acc1 = _edge_kernel(g1, colp, rowp, zeros_nd)[:, :N, :]

    g2 = pl.pallas_call(
        _tc2_body,
        grid=(N // BN,),
        in_specs=[_row_blk(D_H), _row_blk(D_H), _row_blk(D_H), _row_blk(1),
                  _full_blk(D_H, D_H), _full_blk(1, D_H)],
        out_specs=_row_blk(D_H),
        out_shape=jax.ShapeDtypeStruct((N, D_H), jnp.float32),
    )(acc1[0], acc1[1], g1, dinv, W2, b2.reshape(1, D_H))

    acc2 = _edge_kernel(g2, colp, rowp, zeros_nd)[:, :N, :]

    out = pl.pallas_call(
        _tc3_body,
        grid=(N // BN,),
        in_specs=[_row_blk(D_H), _row_blk(D_H), _row_blk(D_H), _row_blk(1),
                  _full_blk(D_H, D_H), _full_blk(1, D_H),
                  _full_blk(D_H, D_OUT), _full_blk(1, D_OUT)],
        out_specs=_row_blk(D_OUT),
        out_shape=jax.ShapeDtypeStruct((N, D_OUT), jnp.float32),
    )(acc2[0], acc2[1], g2, dinv, Wp1, bp1.reshape(1, D_H),
      Wp2, bp2.reshape(1, D_OUT))

    return out
